# baseline (speedup 1.0000x reference)
import functools

import jax
import jax.numpy as jnp
from jax import lax
from jax.experimental import pallas as pl
from jax.experimental.pallas import tpu as pltpu
from jax.experimental.pallas import tpu_sc as plsc

N = 10000
F = 128
H = 8
C = 64
HC = H * C
NEG = 0.2

NC, NS = 2, 16
NW = NC * NS
CH = 128
ROWW = 80
E_PAD = 335872
SLAB = 84
NP = 10240
EPW = E_PAD // NS
NCHUNK = EPW // CH
NB = 400
GRID = N // NB


def _tca_body(x_ref, w_ref, al_ref, ar_ref, h_ref, a_ref):
    h = jnp.dot(x_ref[...], w_ref[...], preferred_element_type=jnp.float32)
    h_ref[...] = h
    h3 = h.reshape(NB, H, C)
    aL = (h3 * al_ref[...][None]).sum(-1)
    aR = (h3 * ar_ref[...][None]).sum(-1)
    a_ref[...] = jnp.concatenate([aL, aR], axis=1)


def _tc_stage_a(x, W1, att_l, att_r):
    return pl.pallas_call(
        _tca_body,
        grid=(GRID,),
        in_specs=[
            pl.BlockSpec((NB, F), lambda i: (i, 0)),
            pl.BlockSpec((F, HC), lambda i: (0, 0)),
            pl.BlockSpec((H, C), lambda i: (0, 0)),
            pl.BlockSpec((H, C), lambda i: (0, 0)),
        ],
        out_specs=[
            pl.BlockSpec((NB, HC), lambda i: (i, 0)),
            pl.BlockSpec((NB, 2 * H), lambda i: (i, 0)),
        ],
        out_shape=[
            jax.ShapeDtypeStruct((N, HC), jnp.float32),
            jax.ShapeDtypeStruct((N, 2 * H), jnp.float32),
        ],
    )(x, W1, att_l, att_r)


def _tcb_body(m0_ref, d0_ref, b_ref, w_ref, al_ref, ar_ref,
              h_ref, a_ref):
    den = d0_ref[...].reshape(NB, H, 1)
    o = m0_ref[...].reshape(NB, H, C) / den
    o = o.reshape(NB, HC) + b_ref[...]
    o = jnp.where(o > 0, o, jnp.exp(o) - 1.0)
    h = jnp.dot(o, w_ref[...], preferred_element_type=jnp.float32)
    h_ref[...] = h
    h3 = h.reshape(NB, H, C)
    aL = (h3 * al_ref[...][None]).sum(-1)
    aR = (h3 * ar_ref[...][None]).sum(-1)
    a_ref[...] = jnp.concatenate([aL, aR], axis=1)


def _tc_stage_b(m0, d0, b1, W2, att_l, att_r):
    return pl.pallas_call(
        _tcb_body,
        grid=(GRID,),
        in_specs=[
            pl.BlockSpec((NB, HC), lambda i: (i, 0)),
            pl.BlockSpec((NB, H), lambda i: (i, 0)),
            pl.BlockSpec((1, HC), lambda i: (0, 0)),
            pl.BlockSpec((HC, HC), lambda i: (0, 0)),
            pl.BlockSpec((H, C), lambda i: (0, 0)),
            pl.BlockSpec((H, C), lambda i: (0, 0)),
        ],
        out_specs=[
            pl.BlockSpec((NB, HC), lambda i: (i, 0)),
            pl.BlockSpec((NB, 2 * H), lambda i: (i, 0)),
        ],
        out_shape=[
            jax.ShapeDtypeStruct((N, HC), jnp.float32),
            jax.ShapeDtypeStruct((N, 2 * H), jnp.float32),
        ],
    )(m0, d0, b1, W2, att_l, att_r)


def _tcc_body(m0_ref, d0_ref, b_ref, o_ref):
    den = d0_ref[...].reshape(NB, H, 1)
    o = m0_ref[...].reshape(NB, H, C) / den
    o = o.mean(axis=1) + b_ref[...]
    m = jnp.max(o, axis=-1, keepdims=True)
    lse = jnp.log(jnp.sum(jnp.exp(o - m), axis=-1, keepdims=True)) + m
    o_ref[...] = o - lse


def _tc_stage_c(m0, d0, b2):
    return pl.pallas_call(
        _tcc_body,
        grid=(GRID,),
        in_specs=[
            pl.BlockSpec((NB, HC), lambda i: (i, 0)),
            pl.BlockSpec((NB, H), lambda i: (i, 0)),
            pl.BlockSpec((1, C), lambda i: (0, 0)),
        ],
        out_specs=pl.BlockSpec((NB, C), lambda i: (i, 0)),
        out_shape=jax.ShapeDtypeStruct((N, C), jnp.float32),
    )(m0, d0, b2)


def _lane_sum(v, lanes):
    for sh in (8, 4, 2, 1):
        idx = jnp.bitwise_xor(lanes, sh)
        v = v + v.at[idx].get(mode="promise_in_bounds")
    return v


def _sc_edge_body(e1, etot, tbl2, atab2, ed3, zrows, out,
                  ed0, ed1, hs0, hd0, hs1, hd1,
                  atv, msg, la, wab, didxS, accum, g0, g1, g2, g3, sc0):
    c = lax.axis_index("c")
    s = lax.axis_index("s")
    wid = s
    tbl = tbl2.at[c]
    rows_per_sub = NP // NS

    pltpu.sync_copy(zrows, accum.at[pl.ds(s * rows_per_sub, rows_per_sub)])
    pltpu.sync_copy(atab2.at[c], atv)
    plsc.subcore_barrier()

    lanes = lax.iota(jnp.int32, 16)

    def load_idx(i, ed):
        pltpu.sync_copy(ed3.at[wid * (NCHUNK + 2) + i], ed)

    def gather(sidx, didx, hs, hd, ghs, ghd):
        pltpu.async_copy(tbl.at[sidx], hs, ghs)
        pltpu.async_copy(tbl.at[didx], hd, ghd)

    def gwait(sidx, didx, hs, hd, ghs, ghd):
        pltpu.make_async_copy(tbl.at[sidx], hs, ghs).wait()
        pltpu.make_async_copy(tbl.at[didx], hd, ghd).wait()

    def step(i, ed, hs, hd, ed_n, hs_n, hd_n, ga, gb, gc, gd):
        sidx = ed.at[0]
        didx = ed.at[1]
        gwait(sidx, didx, hs, hd, ga, gb)
        load_idx(i + 1, ed_n)
        gather(ed_n.at[0], ed_n.at[1], hs_n, hd_n, gc, gd)

        @plsc.parallel_loop(0, CH, unroll=2)
        def dot_body(e):
            pa = hs[e, pl.ds(0, 16)] * hd[e, pl.ds(0, 16)]
            for k in range(1, 4):
                pa = pa + hs[e, pl.ds(k * 16, 16)] * hd[e, pl.ds(k * 16, 16)]
            la[e, :] = pa

        base = wid * EPW + i * CH

        @plsc.parallel_loop(0, CH // 16, unroll=2)
        def grp_body(g):
            ev = g * 16 + lanes
            sv = ed[0, pl.ds(g * 16, 16)]
            dv = ed[1, pl.ds(g * 16, 16)]
            eg = base + ev
            vf = jnp.where(eg < e1, jnp.where(sv != dv, 1.0, 0.0),
                           jnp.where(eg < etot, 1.0, 0.0))
            aL = plsc.load_gather(atv, [sv * 2])
            aR = plsc.load_gather(atv, [dv * 2 + 1])
            lav = plsc.load_gather(la, [ev, lanes * 0])
            for cc in range(1, 16):
                lav = lav + plsc.load_gather(la, [ev, lanes * 0 + cc])
            sa = (aL + aR) / (1.0 + jnp.exp(-lav))
            sa = jnp.maximum(sa, NEG * sa)
            plsc.store_scatter(wab, [ev, lanes * 0], vf * jnp.exp(sa))

        @pl.when(i > 0)
        def _():
            pltpu.make_async_copy(msg, accum.at[didxS], sc0).wait()

        @plsc.parallel_loop(0, CH // 16, unroll=8)
        def cp_body(g):
            didxS[pl.ds(g * 16, 16)] = ed[1, pl.ds(g * 16, 16)]

        @plsc.parallel_loop(0, CH, unroll=2)
        def msg_body(e):
            wrow = wab[e, :]
            w_a = wrow[0]
            for k in range(4):
                msg[e, pl.ds(k * 16, 16)] = hs[e, pl.ds(k * 16, 16)] * w_a
            msg[e, pl.ds(64, 16)] = jnp.where(lanes == 0, w_a, 0.0)

        pltpu.async_copy(msg, accum.at[didxS], sc0, add=True)

    load_idx(0, ed0)
    gather(ed0.at[0], ed0.at[1], hs0, hd0, g0, g1)

    def body(j, carry):
        step(2 * j, ed0, hs0, hd0, ed1, hs1, hd1, g0, g1, g2, g3)
        step(2 * j + 1, ed1, hs1, hd1, ed0, hs0, hd0, g2, g3, g0, g1)
        return carry

    lax.fori_loop(0, NCHUNK // 2, body, 0)

    gwait(ed0.at[0], ed0.at[1], hs0, hd0, g0, g1)
    pltpu.make_async_copy(msg, accum.at[didxS], sc0).wait()

    plsc.subcore_barrier()
    pltpu.sync_copy(accum.at[pl.ds(s * rows_per_sub, rows_per_sub)],
                    out.at[c, pl.ds(s * rows_per_sub, rows_per_sub)])


def _sc_edge_pass(e1, etot, tbl2, atab2, ed3, zrows):
    mesh = plsc.VectorSubcoreMesh(core_axis_name="c", subcore_axis_name="s",
                                  num_cores=NC, num_subcores=NS)
    kern = pl.kernel(
        functools.partial(_sc_edge_body, e1, etot),
        out_type=jax.ShapeDtypeStruct((NC, NP, ROWW), jnp.float32),
        mesh=mesh,
        compiler_params=pltpu.CompilerParams(needs_layout_passes=False,
                                             use_tc_tiling_on_sc=False),
        scratch_types=[
            pltpu.VMEM((2, CH), jnp.int32),
            pltpu.VMEM((2, CH), jnp.int32),
            pltpu.VMEM((CH, C), jnp.float32),
            pltpu.VMEM((CH, C), jnp.float32),
            pltpu.VMEM((CH, C), jnp.float32),
            pltpu.VMEM((CH, C), jnp.float32),
            pltpu.VMEM((2 * N,), jnp.float32),
            pltpu.VMEM((CH, ROWW), jnp.float32),
            pltpu.VMEM((CH, 16), jnp.float32),
            pltpu.VMEM((CH, 16), jnp.float32),
            pltpu.VMEM((CH,), jnp.int32),
            pltpu.VMEM_SHARED((NP, ROWW), jnp.float32),
            pltpu.SemaphoreType.DMA,
            pltpu.SemaphoreType.DMA,
            pltpu.SemaphoreType.DMA,
            pltpu.SemaphoreType.DMA,
            pltpu.SemaphoreType.DMA,
        ],
    )
    return kern(tbl2, atab2, ed3, zrows)


def _edge_phase(e1, etot, h, aLR, ed3, zrows):
    msgs = []
    dens = []
    for p in range(4):
        tbl2 = h[:, 2 * p * C:(2 * p + 2) * C].reshape(N, 2, C).transpose(1, 0, 2)
        atab2 = jnp.stack(
            [jnp.stack([aLR[:, 2 * p], aLR[:, H + 2 * p]], axis=1).reshape(-1),
             jnp.stack([aLR[:, 2 * p + 1], aLR[:, H + 2 * p + 1]],
                       axis=1).reshape(-1)], axis=0)
        part = _sc_edge_pass(e1, etot, tbl2, atab2, ed3, zrows)
        msgs.append(part[0, :N, :C])
        msgs.append(part[1, :N, :C])
        dens.append(part[0, :N, C:C + 1])
        dens.append(part[1, :N, C:C + 1])
    msg = jnp.concatenate(msgs, axis=1)
    den = jnp.concatenate(dens, axis=1)
    return msg, den


def kernel(x, edge_index, W1, att_l1, att_r1, b1, W2, att_l2, att_r2, b2):
    e1 = edge_index.shape[1]
    etot = e1 + N
    loops = jnp.arange(N, dtype=jnp.int32)
    pad = jnp.zeros((E_PAD - etot,), jnp.int32)
    srcs = jnp.concatenate([edge_index[0], loops, pad]).reshape(NS, NCHUNK, CH)
    dsts = jnp.concatenate([edge_index[1], loops, pad]).reshape(NS, NCHUNK, CH)
    padc = jnp.zeros((NS, 2, 2, CH), jnp.int32)
    ed3 = jnp.concatenate(
        [jnp.stack([srcs, dsts], axis=2), padc],
        axis=1).reshape(NS * (NCHUNK + 2), 2, CH)
    zrows = jnp.zeros((NP // NS, ROWW), jnp.float32)

    h1, aLR1 = _tc_stage_a(x, W1, att_l1, att_r1)
    msg1, den1 = _edge_phase(e1, etot, h1, aLR1, ed3, zrows)
    h2, aLR2 = _tc_stage_b(msg1, den1, b1.reshape(1, HC), W2,
                           att_l2, att_r2)
    msg2, den2 = _edge_phase(e1, etot, h2, aLR2, ed3, zrows)
    log_probs = _tc_stage_c(msg2, den2, b2.reshape(1, C))
    att_loss = jnp.array(0.0, dtype=jnp.float32)
    return (log_probs, att_loss)


# speedup vs baseline: 1.1858x; 1.1858x over previous
import functools

import jax
import jax.numpy as jnp
from jax import lax
from jax.experimental import pallas as pl
from jax.experimental.pallas import tpu as pltpu
from jax.experimental.pallas import tpu_sc as plsc

N = 10000
F = 128
H = 8
C = 64
HC = H * C
NEG = 0.2

NC, NS = 2, 16
NW = NC * NS
CH = 128
ROWW = 80
E_PAD = 335872
SLAB = 84
NP = 10240
EPW = E_PAD // NS
NCHUNK = EPW // CH
NB = 400
GRID = N // NB


def _tca_body(x_ref, w_ref, al_ref, ar_ref, h_ref, a_ref):
    h = jnp.dot(x_ref[...], w_ref[...], preferred_element_type=jnp.float32)
    h_ref[...] = h
    h3 = h.reshape(NB, H, C)
    aL = (h3 * al_ref[...][None]).sum(-1)
    aR = (h3 * ar_ref[...][None]).sum(-1)
    a_ref[...] = jnp.concatenate([aL, aR], axis=1)


def _tc_stage_a(x, W1, att_l, att_r):
    return pl.pallas_call(
        _tca_body,
        grid=(GRID,),
        in_specs=[
            pl.BlockSpec((NB, F), lambda i: (i, 0)),
            pl.BlockSpec((F, HC), lambda i: (0, 0)),
            pl.BlockSpec((H, C), lambda i: (0, 0)),
            pl.BlockSpec((H, C), lambda i: (0, 0)),
        ],
        out_specs=[
            pl.BlockSpec((NB, HC), lambda i: (i, 0)),
            pl.BlockSpec((NB, 2 * H), lambda i: (i, 0)),
        ],
        out_shape=[
            jax.ShapeDtypeStruct((N, HC), jnp.float32),
            jax.ShapeDtypeStruct((N, 2 * H), jnp.float32),
        ],
    )(x, W1, att_l, att_r)


def _tcb_body(m0_ref, d0_ref, b_ref, w_ref, al_ref, ar_ref,
              h_ref, a_ref):
    den = d0_ref[...].reshape(NB, H, 1)
    o = m0_ref[...].reshape(NB, H, C) / den
    o = o.reshape(NB, HC) + b_ref[...]
    o = jnp.where(o > 0, o, jnp.exp(o) - 1.0)
    h = jnp.dot(o, w_ref[...], preferred_element_type=jnp.float32)
    h_ref[...] = h
    h3 = h.reshape(NB, H, C)
    aL = (h3 * al_ref[...][None]).sum(-1)
    aR = (h3 * ar_ref[...][None]).sum(-1)
    a_ref[...] = jnp.concatenate([aL, aR], axis=1)


def _tc_stage_b(m0, d0, b1, W2, att_l, att_r):
    return pl.pallas_call(
        _tcb_body,
        grid=(GRID,),
        in_specs=[
            pl.BlockSpec((NB, HC), lambda i: (i, 0)),
            pl.BlockSpec((NB, H), lambda i: (i, 0)),
            pl.BlockSpec((1, HC), lambda i: (0, 0)),
            pl.BlockSpec((HC, HC), lambda i: (0, 0)),
            pl.BlockSpec((H, C), lambda i: (0, 0)),
            pl.BlockSpec((H, C), lambda i: (0, 0)),
        ],
        out_specs=[
            pl.BlockSpec((NB, HC), lambda i: (i, 0)),
            pl.BlockSpec((NB, 2 * H), lambda i: (i, 0)),
        ],
        out_shape=[
            jax.ShapeDtypeStruct((N, HC), jnp.float32),
            jax.ShapeDtypeStruct((N, 2 * H), jnp.float32),
        ],
    )(m0, d0, b1, W2, att_l, att_r)


def _tcc_body(m0_ref, d0_ref, b_ref, o_ref):
    den = d0_ref[...].reshape(NB, H, 1)
    o = m0_ref[...].reshape(NB, H, C) / den
    o = o.mean(axis=1) + b_ref[...]
    m = jnp.max(o, axis=-1, keepdims=True)
    lse = jnp.log(jnp.sum(jnp.exp(o - m), axis=-1, keepdims=True)) + m
    o_ref[...] = o - lse


def _tc_stage_c(m0, d0, b2):
    return pl.pallas_call(
        _tcc_body,
        grid=(GRID,),
        in_specs=[
            pl.BlockSpec((NB, HC), lambda i: (i, 0)),
            pl.BlockSpec((NB, H), lambda i: (i, 0)),
            pl.BlockSpec((1, C), lambda i: (0, 0)),
        ],
        out_specs=pl.BlockSpec((NB, C), lambda i: (i, 0)),
        out_shape=jax.ShapeDtypeStruct((N, C), jnp.float32),
    )(m0, d0, b2)


def _lane_sum(v, lanes):
    for sh in (8, 4, 2, 1):
        idx = jnp.bitwise_xor(lanes, sh)
        v = v + v.at[idx].get(mode="promise_in_bounds")
    return v


def _sc_edge_body(e1, etot, tbl2, atab2, srcs, dsts, zrows, out,
                  sidx0, didx0, sidx1, didx1, hs0, hd0, hs1, hd1,
                  atv, msg, la, wab, didxS, accum, g0, g1, g2, g3, sc0,
                  ix0, ix1):
    c = lax.axis_index("c")
    s = lax.axis_index("s")
    wid = s
    tbl = tbl2.at[c]
    rows_per_sub = NP // NS

    pltpu.sync_copy(zrows, accum.at[pl.ds(s * rows_per_sub, rows_per_sub)])
    pltpu.sync_copy(atab2.at[c], atv)
    plsc.subcore_barrier()

    lanes = lax.iota(jnp.int32, 16)

    def load_idx(i, sidx, didx):
        base = wid * EPW + i * CH
        pltpu.sync_copy(srcs.at[pl.ds(base, CH)], sidx)
        pltpu.sync_copy(dsts.at[pl.ds(base, CH)], didx)

    def load_idx_async(i, sidx, didx, ix):
        base = wid * EPW + i * CH
        pltpu.async_copy(srcs.at[pl.ds(base, CH)], sidx, ix)
        pltpu.async_copy(dsts.at[pl.ds(base, CH)], didx, ix)

    def idx_wait(i, sidx, didx, ix):
        base = wid * EPW + i * CH
        pltpu.make_async_copy(srcs.at[pl.ds(base, CH)], sidx, ix).wait()
        pltpu.make_async_copy(dsts.at[pl.ds(base, CH)], didx, ix).wait()

    def gather(sidx, didx, hs, hd, ghs, ghd):
        pltpu.async_copy(tbl.at[sidx], hs, ghs)
        pltpu.async_copy(tbl.at[didx], hd, ghd)

    def gwait(sidx, didx, hs, hd, ghs, ghd):
        pltpu.make_async_copy(tbl.at[sidx], hs, ghs).wait()
        pltpu.make_async_copy(tbl.at[didx], hd, ghd).wait()

    def step(i, sidx, didx, hs, hd, sidx_n, didx_n, hs_n, hd_n,
             ga, gb, gc, gd, ix, ix_n):
        gwait(sidx, didx, hs, hd, ga, gb)
        idx_wait(i + 1, sidx_n, didx_n, ix_n)
        gather(sidx_n, didx_n, hs_n, hd_n, gc, gd)

        @plsc.parallel_loop(0, CH, unroll=2)
        def dot_body(e):
            pa = hs[e, pl.ds(0, 16)] * hd[e, pl.ds(0, 16)]
            for k in range(1, 4):
                pa = pa + hs[e, pl.ds(k * 16, 16)] * hd[e, pl.ds(k * 16, 16)]
            la[e, :] = pa

        base = wid * EPW + i * CH

        @plsc.parallel_loop(0, CH // 16, unroll=2)
        def grp_body(g):
            ev = g * 16 + lanes
            sv = sidx[pl.ds(g * 16, 16)]
            dv = didx[pl.ds(g * 16, 16)]
            eg = base + ev
            vf = jnp.where(eg < e1, jnp.where(sv != dv, 1.0, 0.0),
                           jnp.where(eg < etot, 1.0, 0.0))
            aL = plsc.load_gather(atv, [sv * 2])
            aR = plsc.load_gather(atv, [dv * 2 + 1])
            lav = plsc.load_gather(la, [ev, lanes * 0])
            for cc in range(1, 16):
                lav = lav + plsc.load_gather(la, [ev, lanes * 0 + cc])
            sa = (aL + aR) / (1.0 + jnp.exp(-lav))
            sa = jnp.maximum(sa, NEG * sa)
            plsc.store_scatter(wab, [ev, lanes * 0], vf * jnp.exp(sa))

        @pl.when(i > 0)
        def _():
            pltpu.make_async_copy(msg, accum.at[didxS], sc0).wait()

        @plsc.parallel_loop(0, CH // 16, unroll=8)
        def cp_body(g):
            didxS[pl.ds(g * 16, 16)] = didx[pl.ds(g * 16, 16)]

        @plsc.parallel_loop(0, CH, unroll=2)
        def msg_body(e):
            wrow = wab[e, :]
            w_a = wrow[0]
            for k in range(4):
                msg[e, pl.ds(k * 16, 16)] = hs[e, pl.ds(k * 16, 16)] * w_a
            msg[e, pl.ds(64, 16)] = jnp.where(lanes == 0, w_a, 0.0)

        pltpu.async_copy(msg, accum.at[didxS], sc0, add=True)

        load_idx_async(i + 2, sidx, didx, ix)

    load_idx(0, sidx0, didx0)
    gather(sidx0, didx0, hs0, hd0, g0, g1)
    load_idx_async(1, sidx1, didx1, ix1)

    def body(j, carry):
        step(2 * j, sidx0, didx0, hs0, hd0, sidx1, didx1, hs1, hd1,
             g0, g1, g2, g3, ix0, ix1)
        step(2 * j + 1, sidx1, didx1, hs1, hd1, sidx0, didx0, hs0, hd0,
             g2, g3, g0, g1, ix1, ix0)
        return carry

    lax.fori_loop(0, NCHUNK // 2, body, 0)

    gwait(sidx0, didx0, hs0, hd0, g0, g1)
    idx_wait(NCHUNK + 1, sidx1, didx1, ix1)
    pltpu.make_async_copy(msg, accum.at[didxS], sc0).wait()

    plsc.subcore_barrier()
    pltpu.sync_copy(accum.at[pl.ds(s * rows_per_sub, rows_per_sub)],
                    out.at[c, pl.ds(s * rows_per_sub, rows_per_sub)])


def _sc_edge_pass(e1, etot, tbl2, atab2, srcs, dsts, zrows):
    mesh = plsc.VectorSubcoreMesh(core_axis_name="c", subcore_axis_name="s",
                                  num_cores=NC, num_subcores=NS)
    kern = pl.kernel(
        functools.partial(_sc_edge_body, e1, etot),
        out_type=jax.ShapeDtypeStruct((NC, NP, ROWW), jnp.float32),
        mesh=mesh,
        compiler_params=pltpu.CompilerParams(needs_layout_passes=False,
                                             use_tc_tiling_on_sc=False),
        scratch_types=[
            pltpu.VMEM((CH,), jnp.int32),
            pltpu.VMEM((CH,), jnp.int32),
            pltpu.VMEM((CH,), jnp.int32),
            pltpu.VMEM((CH,), jnp.int32),
            pltpu.VMEM((CH, C), jnp.float32),
            pltpu.VMEM((CH, C), jnp.float32),
            pltpu.VMEM((CH, C), jnp.float32),
            pltpu.VMEM((CH, C), jnp.float32),
            pltpu.VMEM((2 * N,), jnp.float32),
            pltpu.VMEM((CH, ROWW), jnp.float32),
            pltpu.VMEM((CH, 16), jnp.float32),
            pltpu.VMEM((CH, 16), jnp.float32),
            pltpu.VMEM((CH,), jnp.int32),
            pltpu.VMEM_SHARED((NP, ROWW), jnp.float32),
            pltpu.SemaphoreType.DMA,
            pltpu.SemaphoreType.DMA,
            pltpu.SemaphoreType.DMA,
            pltpu.SemaphoreType.DMA,
            pltpu.SemaphoreType.DMA,
            pltpu.SemaphoreType.DMA,
            pltpu.SemaphoreType.DMA,
        ],
    )
    return kern(tbl2, atab2, srcs, dsts, zrows)


def _edge_phase(e1, etot, h, aLR, srcs, dsts, zrows):
    msgs = []
    dens = []
    for p in range(4):
        tbl2 = h[:, 2 * p * C:(2 * p + 2) * C].reshape(N, 2, C).transpose(1, 0, 2)
        atab2 = jnp.stack(
            [jnp.stack([aLR[:, 2 * p], aLR[:, H + 2 * p]], axis=1).reshape(-1),
             jnp.stack([aLR[:, 2 * p + 1], aLR[:, H + 2 * p + 1]],
                       axis=1).reshape(-1)], axis=0)
        part = _sc_edge_pass(e1, etot, tbl2, atab2, srcs, dsts, zrows)
        msgs.append(part[0, :N, :C])
        msgs.append(part[1, :N, :C])
        dens.append(part[0, :N, C:C + 1])
        dens.append(part[1, :N, C:C + 1])
    msg = jnp.concatenate(msgs, axis=1)
    den = jnp.concatenate(dens, axis=1)
    return msg, den


def kernel(x, edge_index, W1, att_l1, att_r1, b1, W2, att_l2, att_r2, b2):
    e1 = edge_index.shape[1]
    etot = e1 + N
    loops = jnp.arange(N, dtype=jnp.int32)
    pad = jnp.zeros((E_PAD + 2 * CH - etot,), jnp.int32)
    srcs = jnp.concatenate([edge_index[0], loops, pad])
    dsts = jnp.concatenate([edge_index[1], loops, pad])
    zrows = jnp.zeros((NP // NS, ROWW), jnp.float32)

    h1, aLR1 = _tc_stage_a(x, W1, att_l1, att_r1)
    msg1, den1 = _edge_phase(e1, etot, h1, aLR1, srcs, dsts, zrows)
    h2, aLR2 = _tc_stage_b(msg1, den1, b1.reshape(1, HC), W2,
                           att_l2, att_r2)
    msg2, den2 = _edge_phase(e1, etot, h2, aLR2, srcs, dsts, zrows)
    log_probs = _tc_stage_c(msg2, den2, b2.reshape(1, C))
    att_loss = jnp.array(0.0, dtype=jnp.float32)
    return (log_probs, att_loss)
